# Initial kernel scaffold; baseline (speedup 1.0000x reference)
import jax
import jax.numpy as jnp
from jax.experimental import pallas as pl


def kernel(hidden_states, router_w, w1, w2, w3, lora_A, lora_B):
    raise NotImplementedError("write your pallas kernel here")


# speedup vs baseline: 2.7953x; 2.7953x over previous
import functools

import jax
import jax.numpy as jnp
from jax.experimental import pallas as pl
from jax.experimental.pallas import tpu as pltpu

_SCALING = 32.0 / 16.0


def _moe_body(x_ref, rw_ref, acat_ref, bcat_ref, w1_ref, w3_ref, w2_ref,
              out_ref, logits_ref, *, n_exp, rank, tm):
    f = pl.program_id(1)

    @pl.when(f == 0)
    def _routing_and_lora():
        xt = x_ref[...]
        logits = jnp.dot(xt, rw_ref[...].T, preferred_element_type=jnp.float32)
        logits_ref[...] = logits
        lane = jax.lax.broadcasted_iota(jnp.int32, (tm, 128), 1)
        neg = jnp.float32(-jnp.inf)
        lm = jnp.where(lane < n_exp, logits, neg)
        m1 = jnp.max(lm, axis=1, keepdims=True)
        i1 = jnp.min(jnp.where(lm == m1, lane, 1 << 30), axis=1, keepdims=True)
        lm2 = jnp.where(lane == i1, neg, lm)
        m2 = jnp.max(lm2, axis=1, keepdims=True)
        i2 = jnp.min(jnp.where(lm2 == m2, lane, 1 << 30), axis=1, keepdims=True)
        s1 = jax.nn.sigmoid(m1 - m2)
        s2 = 1.0 - s1
        eid = lane // rank
        w_lora = (jnp.where(eid == i1, s1, 0.0)
                  + jnp.where(eid == i2, s2, 0.0)) * _SCALING
        a = jnp.dot(xt, acat_ref[...].T, preferred_element_type=jnp.float32)
        out_ref[...] = jnp.dot(a * w_lora, bcat_ref[...],
                               preferred_element_type=jnp.float32)

    xt = x_ref[...]
    dn = (((1,), (1,)), ((), ()))
    h1 = jax.lax.dot_general(xt, w1_ref[...], dn,
                             preferred_element_type=jnp.float32)
    h3 = jax.lax.dot_general(xt, w3_ref[...], dn,
                             preferred_element_type=jnp.float32)
    h = jax.nn.silu(h1) * h3
    out_ref[...] += jax.lax.dot_general(h, w2_ref[...], dn,
                                        preferred_element_type=jnp.float32)


@functools.partial(jax.jit, static_argnames=("interpret",))
def _moe_block(x, rw_pad, a_cat, b_cat, w1, w2, w3, *, interpret=False):
    t, h = x.shape
    ffn = w1.shape[0]
    tm, fk = 512, 1024
    nt, nf = t // tm, ffn // fk
    body = functools.partial(_moe_body, n_exp=8, rank=16, tm=tm)
    out, logits_pad = pl.pallas_call(
        body,
        grid=(nt, nf),
        in_specs=[
            pl.BlockSpec((tm, h), lambda t_, f_: (t_, 0)),
            pl.BlockSpec((128, h), lambda t_, f_: (0, 0)),
            pl.BlockSpec((128, h), lambda t_, f_: (0, 0)),
            pl.BlockSpec((128, h), lambda t_, f_: (0, 0)),
            pl.BlockSpec((fk, h), lambda t_, f_: (f_, 0)),
            pl.BlockSpec((fk, h), lambda t_, f_: (f_, 0)),
            pl.BlockSpec((h, fk), lambda t_, f_: (0, f_)),
        ],
        out_specs=[
            pl.BlockSpec((tm, h), lambda t_, f_: (t_, 0)),
            pl.BlockSpec((tm, 128), lambda t_, f_: (t_, 0)),
        ],
        out_shape=[
            jax.ShapeDtypeStruct((t, h), jnp.float32),
            jax.ShapeDtypeStruct((t, 128), jnp.float32),
        ],
        compiler_params=pltpu.CompilerParams(
            dimension_semantics=("parallel", "arbitrary"),
        ),
        interpret=interpret,
    )(x, rw_pad, a_cat, b_cat, w1, w3, w2)
    return out, logits_pad


def kernel(hidden_states, router_w, w1, w2, w3, lora_A, lora_B):
    b, s, h = hidden_states.shape
    x = hidden_states.reshape(-1, h)
    e, r = lora_A.shape[0], lora_A.shape[1]
    a_cat = lora_A.reshape(e * r, h)
    b_cat = lora_B.transpose(0, 2, 1).reshape(e * r, h)
    rw_pad = jnp.zeros((128, h), x.dtype).at[:e].set(router_w)
    out, logits_pad = _moe_block(x, rw_pad, a_cat, b_cat, w1, w2, w3)
    return out.reshape(b, s, h), logits_pad[:, :e]


# speedup vs baseline: 3.4199x; 1.2234x over previous
import functools

import jax
import jax.numpy as jnp
from jax.experimental import pallas as pl
from jax.experimental.pallas import tpu as pltpu

_SCALING = 32.0 / 16.0


def _moe_body(x_ref, rw_ref, acat_ref, bcat_ref, w1_ref, w3_ref, w2_ref,
              out_ref, logits_ref, *, n_exp, rank, tm):
    f = pl.program_id(1)

    @pl.when(f == 0)
    def _routing_and_lora():
        xt = x_ref[...]
        logits = jnp.dot(xt, rw_ref[...].T, preferred_element_type=jnp.float32)
        logits_ref[...] = logits
        lane = jax.lax.broadcasted_iota(jnp.int32, (tm, 128), 1)
        neg = jnp.float32(-jnp.inf)
        lm = jnp.where(lane < n_exp, logits, neg)
        m1 = jnp.max(lm, axis=1, keepdims=True)
        i1 = jnp.min(jnp.where(lm == m1, lane, 1 << 30), axis=1, keepdims=True)
        lm2 = jnp.where(lane == i1, neg, lm)
        m2 = jnp.max(lm2, axis=1, keepdims=True)
        i2 = jnp.min(jnp.where(lm2 == m2, lane, 1 << 30), axis=1, keepdims=True)
        s1 = jax.nn.sigmoid(m1 - m2)
        s2 = 1.0 - s1
        eid = lane // rank
        w_lora = (jnp.where(eid == i1, s1, 0.0)
                  + jnp.where(eid == i2, s2, 0.0)) * _SCALING
        a = jnp.dot(xt, acat_ref[...].T, preferred_element_type=jnp.float32)
        out_ref[...] = jnp.dot(a * w_lora, bcat_ref[...],
                               preferred_element_type=jnp.float32)

    xt = x_ref[...]
    dn = (((1,), (1,)), ((), ()))
    h1 = jax.lax.dot_general(xt, w1_ref[...], dn,
                             preferred_element_type=jnp.float32)
    h3 = jax.lax.dot_general(xt, w3_ref[...], dn,
                             preferred_element_type=jnp.float32)
    h = jax.nn.silu(h1) * h3
    out_ref[...] += jax.lax.dot_general(h, w2_ref[...], dn,
                                        preferred_element_type=jnp.float32)


@functools.partial(jax.jit, static_argnames=("interpret",))
def _moe_block(x, rw_pad, a_cat, b_cat, w1, w2, w3, *, interpret=False):
    t, h = x.shape
    ffn = w1.shape[0]
    tm, fk = 1024, 1024
    nt, nf = t // tm, ffn // fk
    body = functools.partial(_moe_body, n_exp=8, rank=16, tm=tm)
    out, logits_pad = pl.pallas_call(
        body,
        grid=(nt, nf),
        in_specs=[
            pl.BlockSpec((tm, h), lambda t_, f_: (t_, 0)),
            pl.BlockSpec((128, h), lambda t_, f_: (0, 0)),
            pl.BlockSpec((128, h), lambda t_, f_: (0, 0)),
            pl.BlockSpec((128, h), lambda t_, f_: (0, 0)),
            pl.BlockSpec((fk, h), lambda t_, f_: (f_, 0)),
            pl.BlockSpec((fk, h), lambda t_, f_: (f_, 0)),
            pl.BlockSpec((h, fk), lambda t_, f_: (0, f_)),
        ],
        out_specs=[
            pl.BlockSpec((tm, h), lambda t_, f_: (t_, 0)),
            pl.BlockSpec((tm, 128), lambda t_, f_: (t_, 0)),
        ],
        out_shape=[
            jax.ShapeDtypeStruct((t, h), jnp.float32),
            jax.ShapeDtypeStruct((t, 128), jnp.float32),
        ],
        compiler_params=pltpu.CompilerParams(
            dimension_semantics=("parallel", "arbitrary"),
        ),
        interpret=interpret,
    )(x, rw_pad, a_cat, b_cat, w1, w3, w2)
    return out, logits_pad


def kernel(hidden_states, router_w, w1, w2, w3, lora_A, lora_B):
    b, s, h = hidden_states.shape
    x = hidden_states.reshape(-1, h)
    e, r = lora_A.shape[0], lora_A.shape[1]
    a_cat = lora_A.reshape(e * r, h)
    b_cat = lora_B.transpose(0, 2, 1).reshape(e * r, h)
    rw_pad = jnp.zeros((128, h), x.dtype).at[:e].set(router_w)
    out, logits_pad = _moe_block(x, rw_pad, a_cat, b_cat, w1, w2, w3)
    return out.reshape(b, s, h), logits_pad[:, :e]
